# Initial kernel scaffold; baseline (speedup 1.0000x reference)
import jax
import jax.numpy as jnp
from jax.experimental import pallas as pl


def kernel(X, rand_vals, w1, b1, w2, b2):
    raise NotImplementedError("write your pallas kernel here")


# speedup vs baseline: 10.3239x; 10.3239x over previous
import jax
import jax.numpy as jnp
from jax.experimental import pallas as pl

ALPHA = 0.5
_ISCLOSE_THR = 1e-8 + 1e-5 * 2.0


def _body(x_ref, r_ref, w1_ref, b1_ref, w2_ref, b2_ref, o_ref):
    t = 2.0 * w1_ref[...] + b1_ref[...]
    h = jnp.tanh(t)
    logits = jnp.dot(h, w2_ref[...],
                     preferred_element_type=jnp.float32) + b2_ref[...]
    m = jnp.max(logits)
    col = jax.lax.broadcasted_iota(jnp.int32, logits.shape, 1)
    p_idx = jnp.min(jnp.where(logits == m, col, jnp.int32(2**30)))
    p = p_idx.astype(jnp.float32)

    x = x_ref[...]
    cond = (jnp.abs(x - 2.0) <= _ISCLOSE_THR) & (r_ref[...] < ALPHA)
    o_ref[...] = jnp.where(cond, p, x)


def kernel(X, rand_vals, w1, b1, w2, b2):
    out = pl.pallas_call(
        _body,
        out_shape=jax.ShapeDtypeStruct(X.shape, X.dtype),
    )(X, rand_vals, w1.reshape(1, -1), b1.reshape(1, -1), w2,
      b2.reshape(1, -1))
    return out
